# baseline (speedup 1.0000x reference)
import functools

import jax
import jax.numpy as jnp
from jax import lax
from jax.experimental import pallas as pl
from jax.experimental.pallas import tpu as pltpu
from jax.experimental.pallas import tpu_sc as plsc

DICT_LEN = 100000
D = 64
NG = 5
NC, NS, L = 2, 16, 16
NW = NC * NS
ROWS_W = DICT_LEN // NW
CHUNK = 125
NCH = ROWS_W // CHUNK


def _sc_retrieve(keys_mem, vals_mem, cue):
    mesh = plsc.VectorSubcoreMesh(core_axis_name="c", subcore_axis_name="s")

    def body(keys_hbm, vals_hbm, cue_hbm, out_d, out_i, out_m, cue_v, keys_v,
             resd_v, resi_v, mrow_v, sem):
        c = lax.axis_index("c")
        s = lax.axis_index("s")
        wid = s * NC + c
        base = wid * ROWS_W

        pltpu.sync_copy(cue_hbm, cue_v)
        cues = [cue_v[pl.ds(q * L, L)] for q in range(D // L)]

        def row_body(r, carry):
            bd, bi = carry
            acc = None
            for q in range(D // L):
                dq = keys_v[r, pl.ds(q * L, L)] - cues[q]
                sq = dq * dq
                acc = sq if acc is None else acc + sq
            tot = jnp.sum(acc)
            take = tot < bd
            bd = jnp.where(take, tot, bd)
            bi = jnp.where(take, r, bi)
            return bd, bi

        best_d = jnp.float32(jnp.inf)
        best_i = jnp.int32(0)
        for ch in range(NCH):
            pltpu.sync_copy(keys_hbm.at[pl.ds(base + ch * CHUNK, CHUNK)],
                            keys_v)
            bd, bi = lax.fori_loop(0, CHUNK, row_body,
                                   (jnp.float32(jnp.inf), jnp.int32(0)))
            take = bd < best_d
            best_i = jnp.where(take, base + ch * CHUNK + bi, best_i)
            best_d = jnp.where(take, bd, best_d)

        resd_v[...] = jnp.full((L,), best_d, jnp.float32)
        resi_v[...] = jnp.full((L,), best_i, jnp.int32)
        pltpu.sync_copy(resd_v, out_d.at[wid])
        pltpu.sync_copy(resi_v, out_i.at[wid])
        pltpu.sync_copy(vals_hbm.at[pl.ds(best_i, 1)], mrow_v)
        pltpu.sync_copy(mrow_v, out_m.at[pl.ds(wid, 1)])

    f = pl.kernel(
        body,
        compiler_params=pltpu.CompilerParams(use_tc_tiling_on_sc=False,
                                             needs_layout_passes=False),
        out_type=(
            jax.ShapeDtypeStruct((NW, L), jnp.float32),
            jax.ShapeDtypeStruct((NW, L), jnp.int32),
            jax.ShapeDtypeStruct((NW, D), jnp.float32),
        ),
        mesh=mesh,
        scratch_types=[
            pltpu.VMEM((D,), jnp.float32),
            pltpu.VMEM((CHUNK, D), jnp.float32),
            pltpu.VMEM((L,), jnp.float32),
            pltpu.VMEM((L,), jnp.int32),
            pltpu.VMEM((1, D), jnp.float32),
            pltpu.SemaphoreType.DMA,
        ],
    )
    return f(keys_mem, vals_mem, cue)


def _tc_body(d2_ref, idx_ref, mrows_ref, x_ref, h_ref, c_ref, wt_ref, b_ref,
             wa_ref, ba_ref, wc_ref, bc_ref, act_ref, val_ref, h_out, c_out):
    d2 = d2_ref[...]
    idx = idx_ref[...]
    mn = jnp.min(d2)
    big = jnp.int32(jnp.iinfo(jnp.int32).max)
    bi = jnp.min(jnp.where(d2 == mn, idx, big))
    rowsel = (idx_ref[:, 0:1] == bi).astype(jnp.float32)
    m_t = jnp.sum(mrows_ref[...] * rowsel, axis=0, keepdims=True)

    xh = jnp.concatenate([x_ref[...], h_ref[...]], axis=1)
    pre = jnp.dot(xh, wt_ref[...], precision=lax.Precision.HIGHEST,
                  preferred_element_type=jnp.float32) + b_ref[...]
    i_t = jax.nn.sigmoid(pre[:, 0 * D:1 * D])
    f_t = jax.nn.sigmoid(pre[:, 1 * D:2 * D])
    g_t = jnp.tanh(pre[:, 2 * D:3 * D])
    o_t = jax.nn.sigmoid(pre[:, 3 * D:4 * D])
    r_t = jax.nn.sigmoid(pre[:, 4 * D:5 * D])
    c_t = f_t * c_ref[...] + i_t * g_t + r_t * m_t
    h_t = o_t * jnp.tanh(c_t)

    logits = jnp.dot(h_t, wa_ref[...], precision=lax.Precision.HIGHEST,
                     preferred_element_type=jnp.float32)
    logits = logits + ba_ref[...]
    act_ref[...] = jax.nn.softmax(logits, axis=-1)
    val_ref[...] = jnp.dot(h_t, wc_ref[...], precision=lax.Precision.HIGHEST,
                           preferred_element_type=jnp.float32) + bc_ref[...]
    h_out[...] = h_t
    c_out[...] = c_t


def kernel(state, p_action, p_reward, timestep, cue, h_prev, c_prev, keys_mem,
           vals_mem, W_ih, W_hh, b_ih, b_hh, W_actor, b_actor, W_critic,
           b_critic):
    d2c, idxc, mrows = _sc_retrieve(keys_mem, vals_mem, cue)

    x_row = jnp.concatenate([state, p_action, p_reward, timestep],
                            axis=-1).reshape(1, D)
    wt = jnp.concatenate([W_ih.T, W_hh.T], axis=0)
    b_row = (b_ih + b_hh).reshape(1, NG * D)

    act, val, h_t, c_t = pl.pallas_call(
        _tc_body,
        out_shape=(
            jax.ShapeDtypeStruct((1, 16), jnp.float32),
            jax.ShapeDtypeStruct((1, 1), jnp.float32),
            jax.ShapeDtypeStruct((1, D), jnp.float32),
            jax.ShapeDtypeStruct((1, D), jnp.float32),
        ),
    )(d2c, idxc, mrows, x_row, h_prev.reshape(1, D), c_prev.reshape(1, D), wt,
      b_row, W_actor.T, b_actor.reshape(1, 16), W_critic.T,
      b_critic.reshape(1, 1))

    return (act.reshape(16), val.reshape(1), h_t.reshape(D), c_t.reshape(D))


# speedup vs baseline: 1.1121x; 1.1121x over previous
import jax
import jax.numpy as jnp
from jax import lax
from jax.experimental import pallas as pl
from jax.experimental.pallas import tpu as pltpu
from jax.experimental.pallas import tpu_sc as plsc

DICT_LEN = 100000
D = 64
NG = 5
NC, NS, L = 2, 16, 16
NW = NC * NS
ROWS_W = DICT_LEN // NW
CHUNK = 125
NCH = ROWS_W // CHUNK
G = 5

_DN = (((1,), (1,)), ((), ()))


def _sc_retrieve(keys_mem, vals_mem, cue):
    mesh = plsc.VectorSubcoreMesh(core_axis_name="c", subcore_axis_name="s")

    def body(keys_hbm, vals_hbm, cue_hbm, out_d, out_i, out_m, cue_v, keys_v,
             resd_v, resi_v, mrow_v, sem0, sem1):
        c = lax.axis_index("c")
        s = lax.axis_index("s")
        wid = s * NC + c
        base = wid * ROWS_W

        pltpu.sync_copy(cue_hbm, cue_v)
        cues = [cue_v[pl.ds(q * L, L)] for q in range(D // L)]
        sems = (sem0, sem1)

        def start(ch):
            return pltpu.async_copy(
                keys_hbm.at[pl.ds(base + ch * CHUNK, CHUNK)],
                keys_v.at[ch % 2], sems[ch % 2])

        handles = {0: start(0)}
        bd = jnp.float32(jnp.inf)
        bi = jnp.int32(0)
        for ch in range(NCH):
            if ch + 1 < NCH:
                handles[ch + 1] = start(ch + 1)
            handles[ch].wait()
            cb = base + ch * CHUNK
            buf = ch % 2

            def group(g, carry, buf=buf, cb=cb):
                gd, gi = carry
                rb = g * G
                for j in range(G):
                    r = rb + j
                    acc = None
                    for q in range(D // L):
                        dq = keys_v[buf, r, pl.ds(q * L, L)] - cues[q]
                        sq = dq * dq
                        acc = sq if acc is None else acc + sq
                    tot = jnp.sum(acc)
                    take = tot < gd
                    gd = jnp.where(take, tot, gd)
                    gi = jnp.where(take, cb + r, gi)
                return gd, gi

            bd, bi = lax.fori_loop(0, CHUNK // G, group, (bd, bi))

        resd_v[...] = jnp.full((L,), bd, jnp.float32)
        resi_v[...] = jnp.full((L,), bi, jnp.int32)
        pltpu.sync_copy(resd_v, out_d.at[wid])
        pltpu.sync_copy(resi_v, out_i.at[wid])
        pltpu.sync_copy(vals_hbm.at[pl.ds(bi, 1)], mrow_v)
        pltpu.sync_copy(mrow_v, out_m.at[pl.ds(wid, 1)])

    f = pl.kernel(
        body,
        compiler_params=pltpu.CompilerParams(use_tc_tiling_on_sc=False,
                                             needs_layout_passes=False),
        out_type=(
            jax.ShapeDtypeStruct((NW, L), jnp.float32),
            jax.ShapeDtypeStruct((NW, L), jnp.int32),
            jax.ShapeDtypeStruct((NW, D), jnp.float32),
        ),
        mesh=mesh,
        scratch_types=[
            pltpu.VMEM((D,), jnp.float32),
            pltpu.VMEM((2, CHUNK, D), jnp.float32),
            pltpu.VMEM((L,), jnp.float32),
            pltpu.VMEM((L,), jnp.int32),
            pltpu.VMEM((1, D), jnp.float32),
            pltpu.SemaphoreType.DMA,
            pltpu.SemaphoreType.DMA,
        ],
    )
    return f(keys_mem, vals_mem, cue)


def _tc_body(d2_ref, idx_ref, mrows_ref, x_ref, h_ref, c_ref, wih_ref,
             whh_ref, bih_ref, bhh_ref, wa_ref, ba_ref, wc_ref, bc_ref,
             act_ref, val_ref, h_out, c_out):
    d2 = d2_ref[...]
    idx = idx_ref[...]
    mn = jnp.min(d2)
    big = jnp.int32(jnp.iinfo(jnp.int32).max)
    bi = jnp.min(jnp.where(d2 == mn, idx, big))
    rowsel = (idx_ref[:, 0:1] == bi).astype(jnp.float32)
    m_t = jnp.sum(mrows_ref[...] * rowsel, axis=0, keepdims=True)

    x = x_ref[...]
    h = h_ref[...]
    pre = (lax.dot_general(x, wih_ref[...], _DN,
                           precision=lax.Precision.HIGHEST,
                           preferred_element_type=jnp.float32) +
           lax.dot_general(h, whh_ref[...], _DN,
                           precision=lax.Precision.HIGHEST,
                           preferred_element_type=jnp.float32) +
           bih_ref[...] + bhh_ref[...])
    i_t = jax.nn.sigmoid(pre[:, 0 * D:1 * D])
    f_t = jax.nn.sigmoid(pre[:, 1 * D:2 * D])
    g_t = jnp.tanh(pre[:, 2 * D:3 * D])
    o_t = jax.nn.sigmoid(pre[:, 3 * D:4 * D])
    r_t = jax.nn.sigmoid(pre[:, 4 * D:5 * D])
    c_t = f_t * c_ref[...] + i_t * g_t + r_t * m_t
    h_t = o_t * jnp.tanh(c_t)

    logits = lax.dot_general(h_t, wa_ref[...], _DN,
                             precision=lax.Precision.HIGHEST,
                             preferred_element_type=jnp.float32)
    logits = logits + ba_ref[...]
    act_ref[...] = jax.nn.softmax(logits, axis=-1)
    val_ref[...] = lax.dot_general(h_t, wc_ref[...], _DN,
                                   precision=lax.Precision.HIGHEST,
                                   preferred_element_type=jnp.float32)
    val_ref[...] += bc_ref[...]
    h_out[...] = h_t
    c_out[...] = c_t


def kernel(state, p_action, p_reward, timestep, cue, h_prev, c_prev, keys_mem,
           vals_mem, W_ih, W_hh, b_ih, b_hh, W_actor, b_actor, W_critic,
           b_critic):
    d2c, idxc, mrows = _sc_retrieve(keys_mem, vals_mem, cue)

    x_row = jnp.concatenate([state, p_action, p_reward, timestep],
                            axis=-1).reshape(1, D)

    act, val, h_t, c_t = pl.pallas_call(
        _tc_body,
        out_shape=(
            jax.ShapeDtypeStruct((1, 16), jnp.float32),
            jax.ShapeDtypeStruct((1, 1), jnp.float32),
            jax.ShapeDtypeStruct((1, D), jnp.float32),
            jax.ShapeDtypeStruct((1, D), jnp.float32),
        ),
    )(d2c, idxc, mrows, x_row, h_prev.reshape(1, D), c_prev.reshape(1, D),
      W_ih, W_hh, b_ih.reshape(1, NG * D), b_hh.reshape(1, NG * D), W_actor,
      b_actor.reshape(1, 16), W_critic, b_critic.reshape(1, 1))

    return (act.reshape(16), val.reshape(1), h_t.reshape(D), c_t.reshape(D))


# speedup vs baseline: 1.3832x; 1.2438x over previous
import jax
import jax.numpy as jnp
from jax import lax
from jax.experimental import pallas as pl
from jax.experimental.pallas import tpu as pltpu
from jax.experimental.pallas import tpu_sc as plsc

DICT_LEN = 100000
D = 64
NG = 5
NC, NS, L = 2, 16, 16
NW = NC * NS
STRIDE = 3120
ROWS_W = DICT_LEN - (NW - 1) * STRIDE
CHUNK = 80
NCH = ROWS_W // CHUNK
G = 5

_DN = (((1,), (1,)), ((), ()))


def _sc_retrieve(keys_mem, vals_mem, cue):
    mesh = plsc.VectorSubcoreMesh(core_axis_name="c", subcore_axis_name="s")

    def body(keys_hbm, vals_hbm, cue_hbm, out_d, out_i, out_m, cue_v, keys_v,
             resd_v, resi_v, gath_v, sem0, sem1, semg):
        c = lax.axis_index("c")
        s = lax.axis_index("s")
        wid = s * NC + c
        base = wid * STRIDE

        pltpu.sync_copy(cue_hbm, cue_v)
        cues = [cue_v[pl.ds(q * L, L)] for q in range(D // L)]
        sems = (sem0, sem1)

        def start(ch):
            return pltpu.async_copy(
                keys_hbm.at[pl.ds(base + ch * CHUNK, CHUNK)],
                keys_v.at[ch % 2], sems[ch % 2])

        handles = {0: start(0)}
        bd = jnp.float32(jnp.inf)
        bi = jnp.int32(0)
        for ch in range(NCH):
            if ch + 1 < NCH:
                handles[ch + 1] = start(ch + 1)
            handles[ch].wait()
            cb = base + ch * CHUNK
            buf = ch % 2

            def group(g, carry, buf=buf, cb=cb):
                gd, gi = carry
                rb = g * G
                for j in range(G):
                    r = rb + j
                    acc = None
                    for q in range(D // L):
                        dq = keys_v[buf, r, pl.ds(q * L, L)] - cues[q]
                        sq = dq * dq
                        acc = sq if acc is None else acc + sq
                    tot = jnp.sum(acc)
                    take = tot < gd
                    gd = jnp.where(take, tot, gd)
                    gi = jnp.where(take, cb + r, gi)
                return gd, gi

            bd, bi = lax.fori_loop(0, CHUNK // G, group, (bd, bi))

        resd_v[0] = jnp.full((L,), bd, jnp.float32)
        resi_v[0] = jnp.full((L,), bi, jnp.int32)
        pltpu.sync_copy(resd_v, out_d.at[wid])
        pltpu.sync_copy(resi_v, out_i.at[wid])
        blk = pl.multiple_of((bi // 8) * 8, 8)
        pltpu.async_copy(vals_hbm.at[pl.ds(blk, 8)], gath_v, semg).wait()
        pltpu.sync_copy(gath_v.at[pl.ds(bi - blk, 1)], out_m.at[wid])

    f = pl.kernel(
        body,
        compiler_params=pltpu.CompilerParams(needs_layout_passes=False),
        out_type=(
            jax.ShapeDtypeStruct((NW, 1, L), jnp.float32),
            jax.ShapeDtypeStruct((NW, 1, L), jnp.int32),
            jax.ShapeDtypeStruct((NW, 1, D), jnp.float32),
        ),
        mesh=mesh,
        scratch_types=[
            pltpu.VMEM((D,), jnp.float32),
            pltpu.VMEM((2, CHUNK, D), jnp.float32),
            pltpu.VMEM((1, L), jnp.float32),
            pltpu.VMEM((1, L), jnp.int32),
            pltpu.VMEM((8, D), jnp.float32),
            pltpu.SemaphoreType.DMA,
            pltpu.SemaphoreType.DMA,
            pltpu.SemaphoreType.DMA,
        ],
    )
    return f(keys_mem, vals_mem, cue)


def _tc_body(d2_ref, idx_ref, mrows_ref, x_ref, h_ref, c_ref, wih_ref,
             whh_ref, bih_ref, bhh_ref, wa_ref, ba_ref, wc_ref, bc_ref,
             act_ref, val_ref, h_out, c_out):
    d2 = d2_ref[...]
    idx = idx_ref[...]
    mn = jnp.min(d2)
    big = jnp.int32(jnp.iinfo(jnp.int32).max)
    bi = jnp.min(jnp.where(d2 == mn, idx, big))
    rowsel = (idx_ref[:, 0:1] == bi).astype(jnp.float32)
    m_t = jnp.sum(mrows_ref[...] * rowsel, axis=0, keepdims=True)

    x = x_ref[...]
    h = h_ref[...]
    pre = (lax.dot_general(x, wih_ref[...], _DN,
                           precision=lax.Precision.HIGHEST,
                           preferred_element_type=jnp.float32) +
           lax.dot_general(h, whh_ref[...], _DN,
                           precision=lax.Precision.HIGHEST,
                           preferred_element_type=jnp.float32) +
           bih_ref[...] + bhh_ref[...])
    i_t = jax.nn.sigmoid(pre[:, 0 * D:1 * D])
    f_t = jax.nn.sigmoid(pre[:, 1 * D:2 * D])
    g_t = jnp.tanh(pre[:, 2 * D:3 * D])
    o_t = jax.nn.sigmoid(pre[:, 3 * D:4 * D])
    r_t = jax.nn.sigmoid(pre[:, 4 * D:5 * D])
    c_t = f_t * c_ref[...] + i_t * g_t + r_t * m_t
    h_t = o_t * jnp.tanh(c_t)

    logits = lax.dot_general(h_t, wa_ref[...], _DN,
                             precision=lax.Precision.HIGHEST,
                             preferred_element_type=jnp.float32)
    logits = logits + ba_ref[...]
    act_ref[...] = jax.nn.softmax(logits, axis=-1)
    val_ref[...] = lax.dot_general(h_t, wc_ref[...], _DN,
                                   precision=lax.Precision.HIGHEST,
                                   preferred_element_type=jnp.float32)
    val_ref[...] += bc_ref[...]
    h_out[...] = h_t
    c_out[...] = c_t


def kernel(state, p_action, p_reward, timestep, cue, h_prev, c_prev, keys_mem,
           vals_mem, W_ih, W_hh, b_ih, b_hh, W_actor, b_actor, W_critic,
           b_critic):
    d2c, idxc, mrows = _sc_retrieve(keys_mem, vals_mem, cue)

    x_row = jnp.concatenate([state, p_action, p_reward, timestep],
                            axis=-1).reshape(1, D)

    act, val, h_t, c_t = pl.pallas_call(
        _tc_body,
        out_shape=(
            jax.ShapeDtypeStruct((1, 16), jnp.float32),
            jax.ShapeDtypeStruct((1, 1), jnp.float32),
            jax.ShapeDtypeStruct((1, D), jnp.float32),
            jax.ShapeDtypeStruct((1, D), jnp.float32),
        ),
    )(d2c.reshape(NW, L), idxc.reshape(NW, L), mrows.reshape(NW, D), x_row,
      h_prev.reshape(1, D), c_prev.reshape(1, D), W_ih, W_hh,
      b_ih.reshape(1, NG * D), b_hh.reshape(1, NG * D), W_actor,
      b_actor.reshape(1, 16), W_critic, b_critic.reshape(1, 1))

    return (act.reshape(16), val.reshape(1), h_t.reshape(D), c_t.reshape(D))


# speedup vs baseline: 1.6945x; 1.2251x over previous
import jax
import jax.numpy as jnp
from jax import lax
from jax.experimental import pallas as pl
from jax.experimental.pallas import tpu as pltpu
from jax.experimental.pallas import tpu_sc as plsc

DICT_LEN = 100000
D = 64
NG = 5
NC, NS, L = 2, 16, 16
NW = NC * NS
STRIDE = 3120
ROWS_W = DICT_LEN - (NW - 1) * STRIDE
CHUNK = 80
NCH = ROWS_W // CHUNK
G = 5

_DN = (((1,), (1,)), ((), ()))


def _sc_retrieve(keys_mem, cue):
    mesh = plsc.VectorSubcoreMesh(core_axis_name="c", subcore_axis_name="s")

    def body(keys_hbm, cue_hbm, out_d, out_i, cue_v, keys_v,
             resd_v, resi_v, sem0, sem1):
        c = lax.axis_index("c")
        s = lax.axis_index("s")
        wid = s * NC + c
        base = wid * STRIDE

        pltpu.sync_copy(cue_hbm, cue_v)
        cues = [cue_v[pl.ds(q * L, L)] for q in range(D // L)]
        sems = (sem0, sem1)

        def start(ch):
            return pltpu.async_copy(
                keys_hbm.at[pl.ds(base + ch * CHUNK, CHUNK)],
                keys_v.at[ch % 2], sems[ch % 2])

        handles = {0: start(0)}
        bd = jnp.float32(jnp.inf)
        bi = jnp.int32(0)
        for ch in range(NCH):
            if ch + 1 < NCH:
                handles[ch + 1] = start(ch + 1)
            handles[ch].wait()
            cb = base + ch * CHUNK
            buf = ch % 2

            def group(g, carry, buf=buf, cb=cb):
                gd, gi = carry
                rb = g * G
                for j in range(G):
                    r = rb + j
                    acc = None
                    for q in range(D // L):
                        dq = keys_v[buf, r, pl.ds(q * L, L)] - cues[q]
                        sq = dq * dq
                        acc = sq if acc is None else acc + sq
                    tot = jnp.sum(acc)
                    take = tot < gd
                    gd = jnp.where(take, tot, gd)
                    gi = jnp.where(take, cb + r, gi)
                return gd, gi

            bd, bi = lax.fori_loop(0, CHUNK // G, group, (bd, bi))

        resd_v[0] = jnp.full((L,), bd, jnp.float32)
        resi_v[0] = jnp.full((L,), bi, jnp.int32)
        pltpu.sync_copy(resd_v, out_d.at[wid])
        pltpu.sync_copy(resi_v, out_i.at[wid])

    f = pl.kernel(
        body,
        compiler_params=pltpu.CompilerParams(needs_layout_passes=False),
        out_type=(
            jax.ShapeDtypeStruct((NW, 1, L), jnp.float32),
            jax.ShapeDtypeStruct((NW, 1, L), jnp.int32),
        ),
        mesh=mesh,
        scratch_types=[
            pltpu.VMEM((D,), jnp.float32),
            pltpu.VMEM((2, CHUNK, D), jnp.float32),
            pltpu.VMEM((1, L), jnp.float32),
            pltpu.VMEM((1, L), jnp.int32),
            pltpu.SemaphoreType.DMA,
            pltpu.SemaphoreType.DMA,
        ],
    )
    return f(keys_mem, cue)


def _tc_body(d2_ref, idx_ref, vals_ref, x_ref, h_ref, c_ref, wih_ref,
             whh_ref, bih_ref, bhh_ref, wa_ref, ba_ref, wc_ref, bc_ref,
             act_ref, val_ref, h_out, c_out, blk_v, semg):
    d2 = d2_ref[...]
    idx = idx_ref[...]
    mn = jnp.min(d2)
    big = jnp.int32(jnp.iinfo(jnp.int32).max)
    bi = jnp.min(jnp.where(d2 == mn, idx, big))
    blk = pl.multiple_of((bi // 8) * 8, 8)
    pltpu.make_async_copy(vals_ref.at[pl.ds(blk, 8)], blk_v, semg).start()
    pltpu.make_async_copy(vals_ref.at[pl.ds(blk, 8)], blk_v, semg).wait()
    rsel = (lax.broadcasted_iota(jnp.int32, (8, 1), 0) == (bi - blk))
    m_t = jnp.sum(blk_v[...] * rsel.astype(jnp.float32), axis=0,
                  keepdims=True)

    x = x_ref[...]
    h = h_ref[...]
    pre = (lax.dot_general(x, wih_ref[...], _DN,
                           precision=lax.Precision.HIGHEST,
                           preferred_element_type=jnp.float32) +
           lax.dot_general(h, whh_ref[...], _DN,
                           precision=lax.Precision.HIGHEST,
                           preferred_element_type=jnp.float32) +
           bih_ref[...] + bhh_ref[...])
    i_t = jax.nn.sigmoid(pre[:, 0 * D:1 * D])
    f_t = jax.nn.sigmoid(pre[:, 1 * D:2 * D])
    g_t = jnp.tanh(pre[:, 2 * D:3 * D])
    o_t = jax.nn.sigmoid(pre[:, 3 * D:4 * D])
    r_t = jax.nn.sigmoid(pre[:, 4 * D:5 * D])
    c_t = f_t * c_ref[...] + i_t * g_t + r_t * m_t
    h_t = o_t * jnp.tanh(c_t)

    logits = lax.dot_general(h_t, wa_ref[...], _DN,
                             precision=lax.Precision.HIGHEST,
                             preferred_element_type=jnp.float32)
    logits = logits + ba_ref[...]
    act_ref[...] = jax.nn.softmax(logits, axis=-1)
    val_ref[...] = lax.dot_general(h_t, wc_ref[...], _DN,
                                   precision=lax.Precision.HIGHEST,
                                   preferred_element_type=jnp.float32)
    val_ref[...] += bc_ref[...]
    h_out[...] = h_t
    c_out[...] = c_t


def kernel(state, p_action, p_reward, timestep, cue, h_prev, c_prev, keys_mem,
           vals_mem, W_ih, W_hh, b_ih, b_hh, W_actor, b_actor, W_critic,
           b_critic):
    d2c, idxc = _sc_retrieve(keys_mem, cue)

    x_row = jnp.concatenate([state, p_action, p_reward, timestep],
                            axis=-1).reshape(1, D)

    act, val, h_t, c_t = pl.pallas_call(
        _tc_body,
        out_shape=(
            jax.ShapeDtypeStruct((1, 16), jnp.float32),
            jax.ShapeDtypeStruct((1, 1), jnp.float32),
            jax.ShapeDtypeStruct((1, D), jnp.float32),
            jax.ShapeDtypeStruct((1, D), jnp.float32),
        ),
        in_specs=[pl.BlockSpec(memory_space=pl.ANY) if i == 2 else
                  pl.BlockSpec(memory_space=pltpu.MemorySpace.VMEM) for i in range(14)],
        scratch_shapes=[pltpu.VMEM((8, D), jnp.float32),
                        pltpu.SemaphoreType.DMA],
    )(d2c.reshape(NW, L), idxc.reshape(NW, L), vals_mem, x_row,
      h_prev.reshape(1, D), c_prev.reshape(1, D), W_ih, W_hh,
      b_ih.reshape(1, NG * D), b_hh.reshape(1, NG * D), W_actor,
      b_actor.reshape(1, 16), W_critic, b_critic.reshape(1, 1))

    return (act.reshape(16), val.reshape(1), h_t.reshape(D), c_t.reshape(D))
